# baseline (device time: 398356 ns/iter reference)
import jax
import jax.numpy as jnp
from jax import lax
from jax.experimental import pallas as pl
from jax.experimental.pallas import tpu as pltpu

T = 4096
D = 2048
CH = 1024
N_CHUNKS = T // CH


def _pair_allreduce(part):

    def body(part_ref, out_ref, comm_ref, a_v, b_v, o_v,
             send_sem, recv_sem, sem_a, sem_b, sem_o):
        my_x = lax.axis_index("x")
        my_y = lax.axis_index("y")
        my_z = lax.axis_index("z")
        peer = (1 - my_x, my_y, my_z)

        barrier_sem = pltpu.get_barrier_semaphore()
        pl.semaphore_signal(
            barrier_sem, inc=1, device_id=peer,
            device_id_type=pl.DeviceIdType.MESH,
        )
        pl.semaphore_wait(barrier_sem, 1)

        rdma = pltpu.make_async_remote_copy(
            src_ref=part_ref,
            dst_ref=comm_ref,
            send_sem=send_sem,
            recv_sem=recv_sem,
            device_id=peer,
            device_id_type=pl.DeviceIdType.MESH,
        )
        rdma.start()
        rdma.wait()

        for c in range(N_CHUNKS):
            rows = pl.ds(c * CH, CH)
            cp_a = pltpu.make_async_copy(part_ref.at[rows], a_v, sem_a)
            cp_b = pltpu.make_async_copy(comm_ref.at[rows], b_v, sem_b)
            cp_a.start()
            cp_b.start()
            cp_a.wait()
            cp_b.wait()
            o_v[...] = a_v[...].astype(jnp.float32) + b_v[...].astype(jnp.float32)
            cp_o = pltpu.make_async_copy(o_v, out_ref.at[rows], sem_o)
            cp_o.start()
            cp_o.wait()

    out, _ = pl.pallas_call(
        body,
        out_shape=[
            jax.ShapeDtypeStruct((T, D), jnp.float32),
            jax.ShapeDtypeStruct((T, D), jnp.bfloat16),
        ],
        in_specs=[pl.BlockSpec(memory_space=pl.MemorySpace.ANY)],
        out_specs=[
            pl.BlockSpec(memory_space=pl.MemorySpace.ANY),
            pl.BlockSpec(memory_space=pl.MemorySpace.ANY),
        ],
        scratch_shapes=[
            pltpu.VMEM((CH, D), jnp.bfloat16),
            pltpu.VMEM((CH, D), jnp.bfloat16),
            pltpu.VMEM((CH, D), jnp.float32),
            pltpu.SemaphoreType.DMA,
            pltpu.SemaphoreType.DMA,
            pltpu.SemaphoreType.DMA,
            pltpu.SemaphoreType.DMA,
            pltpu.SemaphoreType.DMA,
        ],
        compiler_params=pltpu.CompilerParams(collective_id=0),
    )(part)
    return out


def kernel(ids, E):
    v_loc = E.shape[0]
    my_x = lax.axis_index("x")
    local = ids - my_x * v_loc
    mask = (local >= 0) & (local < v_loc)
    safe = jnp.where(mask, local, 0)
    part = jnp.take(E, safe, axis=0)
    part = jnp.where(mask[:, None], part, 0.0).astype(jnp.bfloat16)
    return _pair_allreduce(part)


# device time: 371586 ns/iter; 1.0720x vs baseline; 1.0720x over previous
import jax
import jax.numpy as jnp
from jax import lax
from jax.experimental import pallas as pl
from jax.experimental.pallas import tpu as pltpu

T = 4096
D = 2048
CH = 128
N_CHUNKS = T // CH


def _pair_allreduce(part):

    def body(part_ref, out_ref, comm_ref, a_v, b_v, o_v,
             send_sems, recv_sems, sem_a, sem_b, sem_o):
        my_x = lax.axis_index("x")
        my_y = lax.axis_index("y")
        my_z = lax.axis_index("z")
        peer = (1 - my_x, my_y, my_z)

        barrier_sem = pltpu.get_barrier_semaphore()
        pl.semaphore_signal(
            barrier_sem, inc=1, device_id=peer,
            device_id_type=pl.DeviceIdType.MESH,
        )
        pl.semaphore_wait(barrier_sem, 1)

        rdmas = []
        for c in range(N_CHUNKS):
            rows = pl.ds(c * CH, CH)
            rdma = pltpu.make_async_remote_copy(
                src_ref=part_ref.at[rows],
                dst_ref=comm_ref.at[rows],
                send_sem=send_sems.at[c],
                recv_sem=recv_sems.at[c],
                device_id=peer,
                device_id_type=pl.DeviceIdType.MESH,
            )
            rdma.start()
            rdmas.append(rdma)

        for c in range(N_CHUNKS):
            rows = pl.ds(c * CH, CH)
            rdmas[c].wait_recv()
            cp_a = pltpu.make_async_copy(part_ref.at[rows], a_v, sem_a)
            cp_b = pltpu.make_async_copy(comm_ref.at[rows], b_v, sem_b)
            cp_a.start()
            cp_b.start()
            cp_a.wait()
            cp_b.wait()
            o_v[...] = a_v[...].astype(jnp.float32) + b_v[...].astype(jnp.float32)
            cp_o = pltpu.make_async_copy(o_v, out_ref.at[rows], sem_o)
            cp_o.start()
            cp_o.wait()

        for c in range(N_CHUNKS):
            rdmas[c].wait_send()

    out, _ = pl.pallas_call(
        body,
        out_shape=[
            jax.ShapeDtypeStruct((T, D), jnp.float32),
            jax.ShapeDtypeStruct((T, D), jnp.bfloat16),
        ],
        in_specs=[pl.BlockSpec(memory_space=pl.MemorySpace.ANY)],
        out_specs=[
            pl.BlockSpec(memory_space=pl.MemorySpace.ANY),
            pl.BlockSpec(memory_space=pl.MemorySpace.ANY),
        ],
        scratch_shapes=[
            pltpu.VMEM((CH, D), jnp.bfloat16),
            pltpu.VMEM((CH, D), jnp.bfloat16),
            pltpu.VMEM((CH, D), jnp.float32),
            pltpu.SemaphoreType.DMA((N_CHUNKS,)),
            pltpu.SemaphoreType.DMA((N_CHUNKS,)),
            pltpu.SemaphoreType.DMA,
            pltpu.SemaphoreType.DMA,
            pltpu.SemaphoreType.DMA,
        ],
        compiler_params=pltpu.CompilerParams(collective_id=0),
    )(part)
    return out


def kernel(ids, E):
    v_loc = E.shape[0]
    my_x = lax.axis_index("x")
    local = ids - my_x * v_loc
    mask = (local >= 0) & (local < v_loc)
    safe = jnp.where(mask, local, 0)
    part = jnp.take(E, safe, axis=0)
    part = jnp.where(mask[:, None], part, 0.0).astype(jnp.bfloat16)
    return _pair_allreduce(part)


# device time: 282786 ns/iter; 1.4087x vs baseline; 1.3140x over previous
import jax
import jax.numpy as jnp
from jax import lax
from jax.experimental import pallas as pl
from jax.experimental.pallas import tpu as pltpu

T = 4096
D = 2048
CH = 128
N_CHUNKS = T // CH


def _fused_embed_allreduce(safe_ids, maskf, E):
    def body(ids_ref, mask_ref, e_ref, out_ref, comm_ref, part_ref,
             g_v, g_bf, m_v, a_v, b_v, o_v,
             x_send_sems, x_recv_sems, gsem, msem, psems,
             sem_a, sem_b, sem_o):
        my_x = lax.axis_index("x")
        my_y = lax.axis_index("y")
        my_z = lax.axis_index("z")
        peer = (1 - my_x, my_y, my_z)

        barrier_sem = pltpu.get_barrier_semaphore()
        pl.semaphore_signal(
            barrier_sem, inc=1, device_id=peer,
            device_id_type=pl.DeviceIdType.MESH,
        )
        pl.semaphore_wait(barrier_sem, 1)

        rdmas = []
        part_cps = []
        for c in range(N_CHUNKS):
            rows = pl.ds(c * CH, CH)
            slot = c % 2
            if c >= 2:
                rdmas[c - 2].wait_send()
                part_cps[c - 2].wait()

            cps = []
            for i in range(CH):
                idx = ids_ref[c * CH + i]
                cp = pltpu.make_async_copy(
                    e_ref.at[pl.ds(idx, 1)], g_v.at[pl.ds(i, 1)], gsem)
                cp.start()
                cps.append(cp)
            cpm = pltpu.make_async_copy(mask_ref.at[rows], m_v, msem)
            cpm.start()
            for cp in cps:
                cp.wait()
            cpm.wait()

            g_bf[slot] = (g_v[...] * m_v[...]).astype(jnp.bfloat16)

            rdma = pltpu.make_async_remote_copy(
                src_ref=g_bf.at[slot],
                dst_ref=comm_ref.at[rows],
                send_sem=x_send_sems.at[c],
                recv_sem=x_recv_sems.at[c],
                device_id=peer,
                device_id_type=pl.DeviceIdType.MESH,
            )
            rdma.start()
            rdmas.append(rdma)
            pcp = pltpu.make_async_copy(g_bf.at[slot], part_ref.at[rows],
                                        psems.at[slot])
            pcp.start()
            part_cps.append(pcp)

        for c in range(N_CHUNKS):
            rows = pl.ds(c * CH, CH)
            rdmas[c].wait_recv()
            cp_a = pltpu.make_async_copy(part_ref.at[rows], a_v, sem_a)
            cp_b = pltpu.make_async_copy(comm_ref.at[rows], b_v, sem_b)
            cp_a.start()
            cp_b.start()
            cp_a.wait()
            cp_b.wait()
            o_v[...] = a_v[...].astype(jnp.float32) + b_v[...].astype(jnp.float32)
            cp_o = pltpu.make_async_copy(o_v, out_ref.at[rows], sem_o)
            cp_o.start()
            cp_o.wait()

        for c in range(N_CHUNKS - 2, N_CHUNKS):
            rdmas[c].wait_send()
            part_cps[c].wait()

    out, _, _ = pl.pallas_call(
        body,
        out_shape=[
            jax.ShapeDtypeStruct((T, D), jnp.float32),
            jax.ShapeDtypeStruct((T, D), jnp.bfloat16),
            jax.ShapeDtypeStruct((T, D), jnp.bfloat16),
        ],
        in_specs=[
            pl.BlockSpec(memory_space=pltpu.SMEM),
            pl.BlockSpec(memory_space=pl.MemorySpace.ANY),
            pl.BlockSpec(memory_space=pl.MemorySpace.ANY),
        ],
        out_specs=[
            pl.BlockSpec(memory_space=pl.MemorySpace.ANY),
            pl.BlockSpec(memory_space=pl.MemorySpace.ANY),
            pl.BlockSpec(memory_space=pl.MemorySpace.ANY),
        ],
        scratch_shapes=[
            pltpu.VMEM((CH, D), jnp.float32),
            pltpu.VMEM((2, CH, D), jnp.bfloat16),
            pltpu.VMEM((CH, 1), jnp.float32),
            pltpu.VMEM((CH, D), jnp.bfloat16),
            pltpu.VMEM((CH, D), jnp.bfloat16),
            pltpu.VMEM((CH, D), jnp.float32),
            pltpu.SemaphoreType.DMA((N_CHUNKS,)),
            pltpu.SemaphoreType.DMA((N_CHUNKS,)),
            pltpu.SemaphoreType.DMA,
            pltpu.SemaphoreType.DMA,
            pltpu.SemaphoreType.DMA((2,)),
            pltpu.SemaphoreType.DMA,
            pltpu.SemaphoreType.DMA,
            pltpu.SemaphoreType.DMA,
        ],
        compiler_params=pltpu.CompilerParams(collective_id=0),
    )(safe_ids, maskf, E)
    return out


def kernel(ids, E):
    v_loc = E.shape[0]
    my_x = lax.axis_index("x")
    local = ids - my_x * v_loc
    mask = (local >= 0) & (local < v_loc)
    safe = jnp.where(mask, local, 0).astype(jnp.int32)
    maskf = mask.astype(jnp.float32)[:, None]
    return _fused_embed_allreduce(safe, maskf, E)


# device time: 216772 ns/iter; 1.8377x vs baseline; 1.3045x over previous
import jax
import jax.numpy as jnp
from jax import lax
from jax.experimental import pallas as pl
from jax.experimental.pallas import tpu as pltpu

T = 4096
D = 2048
CH = 128
N_CHUNKS = T // CH
S = 4
LAG = 2


def _fused_embed_allreduce(safe_ids, maskf, E):
    def body(ids_ref, mask_ref, e_ref, out_ref, comm_ref,
             g_v, g_bf, mask_v, b_v, o_v,
             x_send_sems, x_recv_sems, gsems, msem, bsem, osems):
        my_x = lax.axis_index("x")
        my_y = lax.axis_index("y")
        my_z = lax.axis_index("z")
        peer = (1 - my_x, my_y, my_z)

        barrier_sem = pltpu.get_barrier_semaphore()
        pl.semaphore_signal(
            barrier_sem, inc=1, device_id=peer,
            device_id_type=pl.DeviceIdType.MESH,
        )
        pl.semaphore_wait(barrier_sem, 1)

        cpm = pltpu.make_async_copy(mask_ref, mask_v, msem)
        cpm.start()
        cpm.wait()

        def issue_gather(c):
            cps = []
            for i in range(CH):
                idx = ids_ref[c * CH + i]
                cp = pltpu.make_async_copy(
                    e_ref.at[pl.ds(idx, 1)],
                    g_v.at[c % 2].at[pl.ds(i, 1)],
                    gsems.at[c % 2])
                cp.start()
                cps.append(cp)
            return cps

        gather_cps = {0: issue_gather(0)}
        rdmas = []
        bcps = {}
        ocps = {}
        for c in range(N_CHUNKS + LAG):
            d = c - LAG
            if 0 <= d < N_CHUNKS:
                rdmas[d].wait_recv()
                bcp = pltpu.make_async_copy(
                    comm_ref.at[pl.ds(d * CH, CH)], b_v.at[d % 2], bsem)
                bcp.start()
                bcps[d] = bcp

            if c < N_CHUNKS:
                if c + 1 < N_CHUNKS:
                    gather_cps[c + 1] = issue_gather(c + 1)
                for cp in gather_cps.pop(c):
                    cp.wait()
                slot = c % S
                if c >= S:
                    rdmas[c - S].wait_send()
                rows = pl.ds(c * CH, CH)
                g_bf[slot] = (
                    g_v[c % 2] * mask_v[pl.ds(c * CH, CH)]
                ).astype(jnp.bfloat16)
                rdma = pltpu.make_async_remote_copy(
                    src_ref=g_bf.at[slot],
                    dst_ref=comm_ref.at[rows],
                    send_sem=x_send_sems.at[c],
                    recv_sem=x_recv_sems.at[c],
                    device_id=peer,
                    device_id_type=pl.DeviceIdType.MESH,
                )
                rdma.start()
                rdmas.append(rdma)

            if 0 <= d < N_CHUNKS:
                bcps.pop(d).wait()
                if d >= 2:
                    ocps.pop(d - 2).wait()
                o_v[d % 2] = (
                    g_bf[d % S].astype(jnp.float32)
                    + b_v[d % 2].astype(jnp.float32)
                )
                ocp = pltpu.make_async_copy(
                    o_v.at[d % 2], out_ref.at[pl.ds(d * CH, CH)],
                    osems.at[d % 2])
                ocp.start()
                ocps[d] = ocp

        for c in range(N_CHUNKS - S, N_CHUNKS):
            rdmas[c].wait_send()
        for d in list(ocps):
            ocps.pop(d).wait()

    out, _ = pl.pallas_call(
        body,
        out_shape=[
            jax.ShapeDtypeStruct((T, D), jnp.float32),
            jax.ShapeDtypeStruct((T, D), jnp.bfloat16),
        ],
        in_specs=[
            pl.BlockSpec(memory_space=pltpu.SMEM),
            pl.BlockSpec(memory_space=pl.MemorySpace.ANY),
            pl.BlockSpec(memory_space=pl.MemorySpace.ANY),
        ],
        out_specs=[
            pl.BlockSpec(memory_space=pl.MemorySpace.ANY),
            pl.BlockSpec(memory_space=pl.MemorySpace.ANY),
        ],
        scratch_shapes=[
            pltpu.VMEM((2, CH, D), jnp.float32),
            pltpu.VMEM((S, CH, D), jnp.bfloat16),
            pltpu.VMEM((T, 1), jnp.float32),
            pltpu.VMEM((2, CH, D), jnp.bfloat16),
            pltpu.VMEM((2, CH, D), jnp.float32),
            pltpu.SemaphoreType.DMA((N_CHUNKS,)),
            pltpu.SemaphoreType.DMA((N_CHUNKS,)),
            pltpu.SemaphoreType.DMA((2,)),
            pltpu.SemaphoreType.DMA,
            pltpu.SemaphoreType.DMA,
            pltpu.SemaphoreType.DMA((2,)),
        ],
        compiler_params=pltpu.CompilerParams(collective_id=0),
    )(safe_ids, maskf, E)
    return out


def kernel(ids, E):
    v_loc = E.shape[0]
    my_x = lax.axis_index("x")
    local = ids - my_x * v_loc
    mask = (local >= 0) & (local < v_loc)
    safe = jnp.where(mask, local, 0).astype(jnp.int32)
    maskf = mask.astype(jnp.float32)[:, None]
    return _fused_embed_allreduce(safe, maskf, E)


# device time: 159115 ns/iter; 2.5036x vs baseline; 1.3624x over previous
import jax
import jax.numpy as jnp
from jax import lax
from jax.experimental import pallas as pl
from jax.experimental.pallas import tpu as pltpu

T = 4096
D = 2048
HALF = T // 2
CH = 128
NC = HALF // CH
S = 4
LAG = 2
YLAG = 4


def _fused_embed_allreduce(safe_ids, maskf, E):
    def body(ids_ref, mask_ref, e_ref, out_ref, commx_ref, commy_ref,
             g_v, g_bf, mask_v, b_v, s_bf, o_v, yb_v, o2_v,
             xs_sems, xr_sems, ys_sems, yr_sems,
             gsems, msem, bsem, ybsem, osems, o2sems):
        my_x = lax.axis_index("x")
        my_y = lax.axis_index("y")
        my_z = lax.axis_index("z")
        xpeer = (1 - my_x, my_y, my_z)
        ypeer = (my_x, 1 - my_y, my_z)

        h0 = my_y * HALF
        oh0 = (1 - my_y) * HALF

        barrier_sem = pltpu.get_barrier_semaphore()
        for nbr in (xpeer, ypeer):
            pl.semaphore_signal(
                barrier_sem, inc=1, device_id=nbr,
                device_id_type=pl.DeviceIdType.MESH,
            )
        pl.semaphore_wait(barrier_sem, 2)

        cpm = pltpu.make_async_copy(mask_ref.at[pl.ds(h0, HALF)], mask_v, msem)
        cpm.start()
        cpm.wait()

        def issue_gather(c):
            cps = []
            for i in range(CH):
                idx = ids_ref[h0 + c * CH + i]
                cp = pltpu.make_async_copy(
                    e_ref.at[pl.ds(idx, 1)],
                    g_v.at[c % 2].at[pl.ds(i, 1)],
                    gsems.at[c % 2])
                cp.start()
                cps.append(cp)
            return cps

        gather_cps = {0: issue_gather(0)}
        xrdmas = []
        yrdmas = {}
        bcps = {}
        ocps = {}
        o2cps = {}
        for c in range(NC + YLAG):
            d = c - LAG
            if 0 <= d < NC:
                xrdmas[d].wait_recv()
                bcp = pltpu.make_async_copy(
                    commx_ref.at[pl.ds(d * CH, CH)], b_v.at[d % 2], bsem)
                bcp.start()
                bcps[d] = bcp

            if c < NC:
                if c + 1 < NC:
                    gather_cps[c + 1] = issue_gather(c + 1)
                for cp in gather_cps.pop(c):
                    cp.wait()
                slot = c % S
                if c >= S:
                    xrdmas[c - S].wait_send()
                g_bf[slot] = (
                    g_v[c % 2] * mask_v[pl.ds(c * CH, CH)]
                ).astype(jnp.bfloat16)
                rdma = pltpu.make_async_remote_copy(
                    src_ref=g_bf.at[slot],
                    dst_ref=commx_ref.at[pl.ds(c * CH, CH)],
                    send_sem=xs_sems.at[c],
                    recv_sem=xr_sems.at[c],
                    device_id=xpeer,
                    device_id_type=pl.DeviceIdType.MESH,
                )
                rdma.start()
                xrdmas.append(rdma)

            if 0 <= d < NC:
                bcps.pop(d).wait()
                if d >= 2:
                    ocps.pop(d - 2).wait()
                o_v[d % 2] = (
                    g_bf[d % S].astype(jnp.float32)
                    + b_v[d % 2].astype(jnp.float32)
                )
                if d >= S:
                    yrdmas[d - S].wait_send()
                s_bf[d % S] = o_v[d % 2].astype(jnp.bfloat16)
                yrdma = pltpu.make_async_remote_copy(
                    src_ref=s_bf.at[d % S],
                    dst_ref=commy_ref.at[pl.ds(d * CH, CH)],
                    send_sem=ys_sems.at[d],
                    recv_sem=yr_sems.at[d],
                    device_id=ypeer,
                    device_id_type=pl.DeviceIdType.MESH,
                )
                yrdma.start()
                yrdmas[d] = yrdma
                ocp = pltpu.make_async_copy(
                    o_v.at[d % 2], out_ref.at[pl.ds(h0 + d * CH, CH)],
                    osems.at[d % 2])
                ocp.start()
                ocps[d] = ocp

            e = c - YLAG
            if 0 <= e < NC:
                yrdmas[e].wait_recv()
                ybcp = pltpu.make_async_copy(
                    commy_ref.at[pl.ds(e * CH, CH)], yb_v.at[e % 2], ybsem)
                ybcp.start()
                ybcp.wait()
                if e >= 2:
                    o2cps.pop(e - 2).wait()
                o2_v[e % 2] = yb_v[e % 2].astype(jnp.float32)
                o2cp = pltpu.make_async_copy(
                    o2_v.at[e % 2], out_ref.at[pl.ds(oh0 + e * CH, CH)],
                    o2sems.at[e % 2])
                o2cp.start()
                o2cps[e] = o2cp

        for c in range(NC - S, NC):
            xrdmas[c].wait_send()
            yrdmas[c].wait_send()
        for h in list(ocps):
            ocps.pop(h).wait()
        for h in list(o2cps):
            o2cps.pop(h).wait()

    out, _, _ = pl.pallas_call(
        body,
        out_shape=[
            jax.ShapeDtypeStruct((T, D), jnp.float32),
            jax.ShapeDtypeStruct((HALF, D), jnp.bfloat16),
            jax.ShapeDtypeStruct((HALF, D), jnp.bfloat16),
        ],
        in_specs=[
            pl.BlockSpec(memory_space=pltpu.SMEM),
            pl.BlockSpec(memory_space=pl.MemorySpace.ANY),
            pl.BlockSpec(memory_space=pl.MemorySpace.ANY),
        ],
        out_specs=[
            pl.BlockSpec(memory_space=pl.MemorySpace.ANY),
            pl.BlockSpec(memory_space=pl.MemorySpace.ANY),
            pl.BlockSpec(memory_space=pl.MemorySpace.ANY),
        ],
        scratch_shapes=[
            pltpu.VMEM((2, CH, D), jnp.float32),
            pltpu.VMEM((S, CH, D), jnp.bfloat16),
            pltpu.VMEM((HALF, 1), jnp.float32),
            pltpu.VMEM((2, CH, D), jnp.bfloat16),
            pltpu.VMEM((S, CH, D), jnp.bfloat16),
            pltpu.VMEM((2, CH, D), jnp.float32),
            pltpu.VMEM((2, CH, D), jnp.bfloat16),
            pltpu.VMEM((2, CH, D), jnp.float32),
            pltpu.SemaphoreType.DMA((NC,)),
            pltpu.SemaphoreType.DMA((NC,)),
            pltpu.SemaphoreType.DMA((NC,)),
            pltpu.SemaphoreType.DMA((NC,)),
            pltpu.SemaphoreType.DMA((2,)),
            pltpu.SemaphoreType.DMA,
            pltpu.SemaphoreType.DMA,
            pltpu.SemaphoreType.DMA,
            pltpu.SemaphoreType.DMA((2,)),
            pltpu.SemaphoreType.DMA((2,)),
        ],
        compiler_params=pltpu.CompilerParams(collective_id=0),
    )(safe_ids, maskf, E)
    return out


def kernel(ids, E):
    v_loc = E.shape[0]
    my_x = lax.axis_index("x")
    local = ids - my_x * v_loc
    mask = (local >= 0) & (local < v_loc)
    safe = jnp.where(mask, local, 0).astype(jnp.int32)
    maskf = mask.astype(jnp.float32)[:, None]
    return _fused_embed_allreduce(safe, maskf, E)


# device time: 156095 ns/iter; 2.5520x vs baseline; 1.0193x over previous
import jax
import jax.numpy as jnp
from jax import lax
from jax.experimental import pallas as pl
from jax.experimental.pallas import tpu as pltpu

T = 4096
D = 2048
HALF = T // 2
CH = 256
NC = HALF // CH
S = 4
LAG = 2
YLAG = 4


def _fused_embed_allreduce(safe_ids, maskf, E):
    def body(ids_ref, mask_ref, e_ref, out_ref, commx_ref, commy_ref,
             g_v, g_bf, mask_v, b_v, s_bf, o_v, yb_v, o2_v,
             xs_sems, xr_sems, ys_sems, yr_sems,
             gsems, msem, bsem, ybsem, osems, o2sems):
        my_x = lax.axis_index("x")
        my_y = lax.axis_index("y")
        my_z = lax.axis_index("z")
        xpeer = (1 - my_x, my_y, my_z)
        ypeer = (my_x, 1 - my_y, my_z)

        h0 = my_y * HALF
        oh0 = (1 - my_y) * HALF

        barrier_sem = pltpu.get_barrier_semaphore()
        for nbr in (xpeer, ypeer):
            pl.semaphore_signal(
                barrier_sem, inc=1, device_id=nbr,
                device_id_type=pl.DeviceIdType.MESH,
            )
        pl.semaphore_wait(barrier_sem, 2)

        cpm = pltpu.make_async_copy(mask_ref.at[pl.ds(h0, HALF)], mask_v, msem)
        cpm.start()
        cpm.wait()

        def issue_gather(c):
            cps = []
            for i in range(CH):
                idx = ids_ref[h0 + c * CH + i]
                cp = pltpu.make_async_copy(
                    e_ref.at[pl.ds(idx, 1)],
                    g_v.at[c % 2].at[pl.ds(i, 1)],
                    gsems.at[c % 2])
                cp.start()
                cps.append(cp)
            return cps

        gather_cps = {0: issue_gather(0)}
        xrdmas = []
        yrdmas = {}
        bcps = {}
        ocps = {}
        o2cps = {}
        ybcps = {}
        for c in range(NC + YLAG):
            d = c - LAG
            if 0 <= d < NC:
                xrdmas[d].wait_recv()
                bcp = pltpu.make_async_copy(
                    commx_ref.at[pl.ds(d * CH, CH)], b_v.at[d % 2], bsem)
                bcp.start()
                bcps[d] = bcp

            if c < NC:
                if c + 1 < NC:
                    gather_cps[c + 1] = issue_gather(c + 1)
                for cp in gather_cps.pop(c):
                    cp.wait()
                slot = c % S
                if c >= S:
                    xrdmas[c - S].wait_send()
                g_bf[slot] = (
                    g_v[c % 2] * mask_v[pl.ds(c * CH, CH)]
                ).astype(jnp.bfloat16)
                rdma = pltpu.make_async_remote_copy(
                    src_ref=g_bf.at[slot],
                    dst_ref=commx_ref.at[pl.ds(c * CH, CH)],
                    send_sem=xs_sems.at[c],
                    recv_sem=xr_sems.at[c],
                    device_id=xpeer,
                    device_id_type=pl.DeviceIdType.MESH,
                )
                rdma.start()
                xrdmas.append(rdma)

            if 0 <= d < NC:
                bcps.pop(d).wait()
                if d >= 2:
                    ocps.pop(d - 2).wait()
                o_v[d % 2] = (
                    g_bf[d % S].astype(jnp.float32)
                    + b_v[d % 2].astype(jnp.float32)
                )
                if d >= S:
                    yrdmas[d - S].wait_send()
                s_bf[d % S] = o_v[d % 2].astype(jnp.bfloat16)
                yrdma = pltpu.make_async_remote_copy(
                    src_ref=s_bf.at[d % S],
                    dst_ref=commy_ref.at[pl.ds(d * CH, CH)],
                    send_sem=ys_sems.at[d],
                    recv_sem=yr_sems.at[d],
                    device_id=ypeer,
                    device_id_type=pl.DeviceIdType.MESH,
                )
                yrdma.start()
                yrdmas[d] = yrdma
                ocp = pltpu.make_async_copy(
                    o_v.at[d % 2], out_ref.at[pl.ds(h0 + d * CH, CH)],
                    osems.at[d % 2])
                ocp.start()
                ocps[d] = ocp

            ep = c - YLAG + 1
            if 0 <= ep < NC:
                yrdmas[ep].wait_recv()
                ybcp = pltpu.make_async_copy(
                    commy_ref.at[pl.ds(ep * CH, CH)], yb_v.at[ep % 2], ybsem)
                ybcp.start()
                ybcps[ep] = ybcp
            e = c - YLAG
            if 0 <= e < NC:
                ybcps.pop(e).wait()
                if e >= 2:
                    o2cps.pop(e - 2).wait()
                o2_v[e % 2] = yb_v[e % 2].astype(jnp.float32)
                o2cp = pltpu.make_async_copy(
                    o2_v.at[e % 2], out_ref.at[pl.ds(oh0 + e * CH, CH)],
                    o2sems.at[e % 2])
                o2cp.start()
                o2cps[e] = o2cp

        for c in range(NC - S, NC):
            xrdmas[c].wait_send()
            yrdmas[c].wait_send()
        for h in list(ocps):
            ocps.pop(h).wait()
        for h in list(o2cps):
            o2cps.pop(h).wait()

    out, _, _ = pl.pallas_call(
        body,
        out_shape=[
            jax.ShapeDtypeStruct((T, D), jnp.float32),
            jax.ShapeDtypeStruct((HALF, D), jnp.bfloat16),
            jax.ShapeDtypeStruct((HALF, D), jnp.bfloat16),
        ],
        in_specs=[
            pl.BlockSpec(memory_space=pltpu.SMEM),
            pl.BlockSpec(memory_space=pl.MemorySpace.ANY),
            pl.BlockSpec(memory_space=pl.MemorySpace.ANY),
        ],
        out_specs=[
            pl.BlockSpec(memory_space=pl.MemorySpace.ANY),
            pl.BlockSpec(memory_space=pl.MemorySpace.ANY),
            pl.BlockSpec(memory_space=pl.MemorySpace.ANY),
        ],
        scratch_shapes=[
            pltpu.VMEM((2, CH, D), jnp.float32),
            pltpu.VMEM((S, CH, D), jnp.bfloat16),
            pltpu.VMEM((HALF, 1), jnp.float32),
            pltpu.VMEM((2, CH, D), jnp.bfloat16),
            pltpu.VMEM((S, CH, D), jnp.bfloat16),
            pltpu.VMEM((2, CH, D), jnp.float32),
            pltpu.VMEM((2, CH, D), jnp.bfloat16),
            pltpu.VMEM((2, CH, D), jnp.float32),
            pltpu.SemaphoreType.DMA((NC,)),
            pltpu.SemaphoreType.DMA((NC,)),
            pltpu.SemaphoreType.DMA((NC,)),
            pltpu.SemaphoreType.DMA((NC,)),
            pltpu.SemaphoreType.DMA((2,)),
            pltpu.SemaphoreType.DMA,
            pltpu.SemaphoreType.DMA,
            pltpu.SemaphoreType.DMA,
            pltpu.SemaphoreType.DMA((2,)),
            pltpu.SemaphoreType.DMA((2,)),
        ],
        compiler_params=pltpu.CompilerParams(collective_id=0),
    )(safe_ids, maskf, E)
    return out


def kernel(ids, E):
    v_loc = E.shape[0]
    my_x = lax.axis_index("x")
    local = ids - my_x * v_loc
    mask = (local >= 0) & (local < v_loc)
    safe = jnp.where(mask, local, 0).astype(jnp.int32)
    maskf = mask.astype(jnp.float32)[:, None]
    return _fused_embed_allreduce(safe, maskf, E)


# device time: 146286 ns/iter; 2.7231x vs baseline; 1.0671x over previous
import jax
import jax.numpy as jnp
from jax import lax
from jax.experimental import pallas as pl
from jax.experimental.pallas import tpu as pltpu

T = 4096
D = 2048
HALF = T // 2
CH = 256
NC = HALF // CH
S = 4
LAG = 2
YLAG = 4


def _fused_embed_allreduce(safe_ids, maskf, E):
    def body(ids_ref, mask_ref, e_ref, out_ref, commx_ref,
             g_v, g_bf, mask_v, b_v, s_bf,
             xs_sems, xr_sems, ys_sems, yr_sems,
             gsems, msem, bsem, ssems):
        my_x = lax.axis_index("x")
        my_y = lax.axis_index("y")
        my_z = lax.axis_index("z")
        xpeer = (1 - my_x, my_y, my_z)
        ypeer = (my_x, 1 - my_y, my_z)

        h0 = my_y * HALF

        barrier_sem = pltpu.get_barrier_semaphore()
        for nbr in (xpeer, ypeer):
            pl.semaphore_signal(
                barrier_sem, inc=1, device_id=nbr,
                device_id_type=pl.DeviceIdType.MESH,
            )
        pl.semaphore_wait(barrier_sem, 2)

        cpm = pltpu.make_async_copy(mask_ref.at[pl.ds(h0, HALF)], mask_v, msem)
        cpm.start()
        cpm.wait()

        def issue_gather(c):
            cps = []
            for i in range(CH):
                idx = ids_ref[h0 + c * CH + i]
                cp = pltpu.make_async_copy(
                    e_ref.at[pl.ds(idx, 1)],
                    g_v.at[c % 2].at[pl.ds(i, 1)],
                    gsems.at[c % 2])
                cp.start()
                cps.append(cp)
            return cps

        gather_cps = {0: issue_gather(0)}
        xrdmas = []
        yrdmas = {}
        bcps = {}
        scps = {}
        for c in range(NC + YLAG):
            d = c - LAG
            if 0 <= d < NC:
                xrdmas[d].wait_recv()
                bcp = pltpu.make_async_copy(
                    commx_ref.at[pl.ds(d * CH, CH)], b_v.at[d % 2], bsem)
                bcp.start()
                bcps[d] = bcp

            if c < NC:
                if c + 1 < NC:
                    gather_cps[c + 1] = issue_gather(c + 1)
                for cp in gather_cps.pop(c):
                    cp.wait()
                slot = c % S
                if c >= S:
                    xrdmas[c - S].wait_send()
                g_bf[slot] = (
                    g_v[c % 2] * mask_v[pl.ds(c * CH, CH)]
                ).astype(jnp.bfloat16)
                rdma = pltpu.make_async_remote_copy(
                    src_ref=g_bf.at[slot],
                    dst_ref=commx_ref.at[pl.ds(c * CH, CH)],
                    send_sem=xs_sems.at[c],
                    recv_sem=xr_sems.at[c],
                    device_id=xpeer,
                    device_id_type=pl.DeviceIdType.MESH,
                )
                rdma.start()
                xrdmas.append(rdma)

            if 0 <= d < NC:
                bcps.pop(d).wait()
                if d >= S:
                    yrdmas[d - S].wait_send()
                    scps.pop(d - S).wait()
                rows = pl.ds(h0 + d * CH, CH)
                s_bf[d % S] = (
                    g_bf[d % S].astype(jnp.float32)
                    + b_v[d % 2].astype(jnp.float32)
                ).astype(jnp.bfloat16)
                yrdma = pltpu.make_async_remote_copy(
                    src_ref=s_bf.at[d % S],
                    dst_ref=out_ref.at[rows],
                    send_sem=ys_sems.at[d],
                    recv_sem=yr_sems.at[d],
                    device_id=ypeer,
                    device_id_type=pl.DeviceIdType.MESH,
                )
                yrdma.start()
                yrdmas[d] = yrdma
                scp = pltpu.make_async_copy(
                    s_bf.at[d % S], out_ref.at[rows], ssems.at[d % S])
                scp.start()
                scps[d] = scp

            e = c - YLAG
            if 0 <= e < NC:
                yrdmas[e].wait_recv()

        for c in range(NC - S, NC):
            xrdmas[c].wait_send()
            yrdmas[c].wait_send()
        for h in list(scps):
            scps.pop(h).wait()

    out, _ = pl.pallas_call(
        body,
        out_shape=[
            jax.ShapeDtypeStruct((T, D), jnp.bfloat16),
            jax.ShapeDtypeStruct((HALF, D), jnp.bfloat16),
        ],
        in_specs=[
            pl.BlockSpec(memory_space=pltpu.SMEM),
            pl.BlockSpec(memory_space=pl.MemorySpace.ANY),
            pl.BlockSpec(memory_space=pl.MemorySpace.ANY),
        ],
        out_specs=[
            pl.BlockSpec(memory_space=pl.MemorySpace.ANY),
            pl.BlockSpec(memory_space=pl.MemorySpace.ANY),
        ],
        scratch_shapes=[
            pltpu.VMEM((2, CH, D), jnp.float32),
            pltpu.VMEM((S, CH, D), jnp.bfloat16),
            pltpu.VMEM((HALF, 1), jnp.float32),
            pltpu.VMEM((2, CH, D), jnp.bfloat16),
            pltpu.VMEM((S, CH, D), jnp.bfloat16),
            pltpu.SemaphoreType.DMA((NC,)),
            pltpu.SemaphoreType.DMA((NC,)),
            pltpu.SemaphoreType.DMA((NC,)),
            pltpu.SemaphoreType.DMA((NC,)),
            pltpu.SemaphoreType.DMA((2,)),
            pltpu.SemaphoreType.DMA,
            pltpu.SemaphoreType.DMA,
            pltpu.SemaphoreType.DMA((S,)),
        ],
        compiler_params=pltpu.CompilerParams(collective_id=0),
    )(safe_ids, maskf, E)
    return out


def kernel(ids, E):
    v_loc = E.shape[0]
    my_x = lax.axis_index("x")
    local = ids - my_x * v_loc
    mask = (local >= 0) & (local < v_loc)
    safe = jnp.where(mask, local, 0).astype(jnp.int32)
    maskf = mask.astype(jnp.float32)[:, None]
    return _fused_embed_allreduce(safe, maskf, E).astype(jnp.float32)


# device time: 140882 ns/iter; 2.8276x vs baseline; 1.0384x over previous
import jax
import jax.numpy as jnp
from jax import lax
from jax.experimental import pallas as pl
from jax.experimental.pallas import tpu as pltpu

T = 4096
D = 2048
HALF = T // 2
CH = 256
NC = HALF // CH
S = 4
LAG = 2
YLAG = 4


def _fused_embed_allreduce(safe_ids, maskf, E):
    def body(ids_ref, mask_ref, e_ref, out_ref, commx_ref,
             g_v, g_bf, mask_v, b_v, s_bf,
             xs_sems, xr_sems, ys_sems, yr_sems,
             gsems, msem, bsem, ssems):
        my_x = lax.axis_index("x")
        my_y = lax.axis_index("y")
        my_z = lax.axis_index("z")
        xpeer = (1 - my_x, my_y, my_z)
        ypeer = (my_x, 1 - my_y, my_z)

        h0 = my_y * HALF

        barrier_sem = pltpu.get_barrier_semaphore()
        for nbr in (xpeer, ypeer):
            pl.semaphore_signal(
                barrier_sem, inc=1, device_id=nbr,
                device_id_type=pl.DeviceIdType.MESH,
            )
        pl.semaphore_wait(barrier_sem, 2)

        cpm = pltpu.make_async_copy(mask_ref.at[pl.ds(h0, HALF)], mask_v, msem)
        cpm.start()
        cpm.wait()

        def issue_gather(c):
            cps = []
            for i in range(CH):
                idx = ids_ref[h0 + c * CH + i]
                cp = pltpu.make_async_copy(
                    e_ref.at[pl.ds(idx, 1)],
                    g_v.at[c % 2].at[pl.ds(i, 1)],
                    gsems.at[c % 2])
                cp.start()
                cps.append(cp)
            return cps

        gather_cps = {0: issue_gather(0)}
        xrdmas = []
        yrdmas = {}
        bcps = {}
        scps = {}
        for c in range(NC + YLAG):
            d = c - LAG
            if 0 <= d < NC:
                xrdmas[d].wait_recv()
                bcp = pltpu.make_async_copy(
                    commx_ref.at[pl.ds(d * CH, CH)], b_v.at[d % 2], bsem)
                bcp.start()
                bcps[d] = bcp

            if c < NC:
                if c + 1 < NC:
                    gather_cps[c + 1] = issue_gather(c + 1)
                for cp in gather_cps.pop(c):
                    cp.wait()
                slot = c % S
                if c >= S:
                    xrdmas[c - S].wait_send()
                g_bf[slot] = (
                    g_v[c % 2] * mask_v[pl.ds(c * CH, CH)]
                ).astype(jnp.bfloat16)
                rdma = pltpu.make_async_remote_copy(
                    src_ref=g_bf.at[slot],
                    dst_ref=commx_ref.at[pl.ds(c * CH, CH)],
                    send_sem=xs_sems.at[c],
                    recv_sem=xr_sems.at[c],
                    device_id=xpeer,
                    device_id_type=pl.DeviceIdType.MESH,
                )
                rdma.start()
                xrdmas.append(rdma)

            if 0 <= d < NC:
                bcps.pop(d).wait()
                if d >= S:
                    yrdmas[d - S].wait_send()
                    scps.pop(d - S).wait()
                rows = pl.ds(h0 + d * CH, CH)
                s_bf[d % S] = (
                    g_bf[d % S].astype(jnp.float32)
                    + b_v[d % 2].astype(jnp.float32)
                ).astype(jnp.bfloat16)
                yrdma = pltpu.make_async_remote_copy(
                    src_ref=s_bf.at[d % S],
                    dst_ref=out_ref.at[rows],
                    send_sem=ys_sems.at[d],
                    recv_sem=yr_sems.at[d],
                    device_id=ypeer,
                    device_id_type=pl.DeviceIdType.MESH,
                )
                yrdma.start()
                yrdmas[d] = yrdma
                scp = pltpu.make_async_copy(
                    s_bf.at[d % S], out_ref.at[rows], ssems.at[d % S])
                scp.start()
                scps[d] = scp

            e = c - YLAG
            if 0 <= e < NC:
                yrdmas[e].wait_recv()

        for c in range(NC - S, NC):
            xrdmas[c].wait_send()
            yrdmas[c].wait_send()
        for h in list(scps):
            scps.pop(h).wait()

    out, _ = pl.pallas_call(
        body,
        out_shape=[
            jax.ShapeDtypeStruct((T, D), jnp.bfloat16),
            jax.ShapeDtypeStruct((HALF, D), jnp.bfloat16),
        ],
        in_specs=[
            pl.BlockSpec(memory_space=pltpu.SMEM),
            pl.BlockSpec(memory_space=pl.MemorySpace.ANY),
            pl.BlockSpec(memory_space=pl.MemorySpace.ANY),
        ],
        out_specs=[
            pl.BlockSpec(memory_space=pl.MemorySpace.ANY),
            pl.BlockSpec(memory_space=pl.MemorySpace.ANY),
        ],
        scratch_shapes=[
            pltpu.VMEM((2, CH, D), jnp.float32),
            pltpu.VMEM((S, CH, D), jnp.bfloat16),
            pltpu.VMEM((HALF, 1), jnp.float32),
            pltpu.VMEM((2, CH, D), jnp.bfloat16),
            pltpu.VMEM((S, CH, D), jnp.bfloat16),
            pltpu.SemaphoreType.DMA((NC,)),
            pltpu.SemaphoreType.DMA((NC,)),
            pltpu.SemaphoreType.DMA((NC,)),
            pltpu.SemaphoreType.DMA((NC,)),
            pltpu.SemaphoreType.DMA((2,)),
            pltpu.SemaphoreType.DMA,
            pltpu.SemaphoreType.DMA,
            pltpu.SemaphoreType.DMA((S,)),
        ],
        compiler_params=pltpu.CompilerParams(collective_id=0),
    )(safe_ids, maskf, E)
    return out


def kernel(ids, E):
    v_loc = E.shape[0]
    my_x = lax.axis_index("x")
    local = ids - my_x * v_loc
    mask = (local >= 0) & (local < v_loc)
    safe = jnp.where(mask, local, 0).astype(jnp.int32)
    maskf = mask.astype(jnp.float32)[:, None]
    return _fused_embed_allreduce(safe, maskf, E)


# device time: 118767 ns/iter; 3.3541x vs baseline; 1.1862x over previous
import jax
import jax.numpy as jnp
from jax import lax
from jax.experimental import pallas as pl
from jax.experimental.pallas import tpu as pltpu

T = 4096
D = 2048
Q = T // 4
CH = 256
NC = Q // CH
HC = CH // 2
LAG = 2
ALAG = 4


def _fused_embed_allreduce(safe_ids, maskf, E):
    def body(ids_ref, mask_ref, e_ref, out_ref, commx_ref,
             g_v, g_bf, mask_v, b_v, s_bf,
             xs_sems, xr_sems, ys_sems, yr_sems, zs_sems, zr_sems,
             f1s_sems, f1r_sems, f2s_sems, f2r_sems,
             gsems, msem, bsem, ssems):
        my_x = lax.axis_index("x")
        my_y = lax.axis_index("y")
        my_z = lax.axis_index("z")
        zb = my_z % 2
        zpeer_z = my_z + 1 - 2 * zb
        xpeer = (1 - my_x, my_y, my_z)
        ypeer = (my_x, 1 - my_y, my_z)
        zpeer = (my_x, my_y, zpeer_z)

        p_me = 2 * my_y + zb
        p_y = 2 * (1 - my_y) + zb
        p_z = 2 * my_y + (1 - zb)
        q0 = p_me * Q

        barrier_sem = pltpu.get_barrier_semaphore()
        for nbr in (xpeer, ypeer, zpeer):
            pl.semaphore_signal(
                barrier_sem, inc=1, device_id=nbr,
                device_id_type=pl.DeviceIdType.MESH,
            )
        pl.semaphore_wait(barrier_sem, 3)

        cpm = pltpu.make_async_copy(mask_ref.at[pl.ds(q0, Q)], mask_v, msem)
        cpm.start()
        cpm.wait()

        def issue_gather(c):
            cps = []
            for i in range(CH):
                idx = ids_ref[q0 + c * CH + i]
                cp = pltpu.make_async_copy(
                    e_ref.at[pl.ds(idx, 1)],
                    g_v.at[c % 2].at[pl.ds(i, 1)],
                    gsems.at[c % 2])
                cp.start()
                cps.append(cp)
            return cps

        gather_cps = {0: issue_gather(0)}
        xrdmas = []
        yrdmas = {}
        zrdmas = {}
        f1rdmas = {}
        f2rdmas = {}
        bcps = {}
        scps = {}
        for c in range(NC + ALAG):
            d = c - LAG
            if 0 <= d < NC:
                xrdmas[d].wait_recv()
                bcp = pltpu.make_async_copy(
                    commx_ref.at[pl.ds(d * CH, CH)], b_v.at[d % 2], bsem)
                bcp.start()
                bcps[d] = bcp

            if c < NC:
                if c + 1 < NC:
                    gather_cps[c + 1] = issue_gather(c + 1)
                for cp in gather_cps.pop(c):
                    cp.wait()
                g_bf[c] = (
                    g_v[c % 2] * mask_v[pl.ds(c * CH, CH)]
                ).astype(jnp.bfloat16)
                rdma = pltpu.make_async_remote_copy(
                    src_ref=g_bf.at[c],
                    dst_ref=commx_ref.at[pl.ds(c * CH, CH)],
                    send_sem=xs_sems.at[c],
                    recv_sem=xr_sems.at[c],
                    device_id=xpeer,
                    device_id_type=pl.DeviceIdType.MESH,
                )
                rdma.start()
                xrdmas.append(rdma)

            if 0 <= d < NC:
                bcps.pop(d).wait()
                rows = pl.ds(q0 + d * CH, CH)
                s_bf[d] = (
                    g_bf[d].astype(jnp.float32)
                    + b_v[d % 2].astype(jnp.float32)
                ).astype(jnp.bfloat16)
                for peer, ss, rr, book in (
                    (ypeer, ys_sems, yr_sems, yrdmas),
                    (zpeer, zs_sems, zr_sems, zrdmas),
                ):
                    prdma = pltpu.make_async_remote_copy(
                        src_ref=s_bf.at[d],
                        dst_ref=out_ref.at[rows],
                        send_sem=ss.at[d],
                        recv_sem=rr.at[d],
                        device_id=peer,
                        device_id_type=pl.DeviceIdType.MESH,
                    )
                    prdma.start()
                    book[d] = prdma
                scp = pltpu.make_async_copy(
                    s_bf.at[d], out_ref.at[rows], ssems.at[d])
                scp.start()
                scps[d] = scp

            e = c - ALAG
            if 0 <= e < NC:
                yrdmas[e].wait_recv()
                base1 = p_y * Q + e * CH + HC
                f1 = pltpu.make_async_remote_copy(
                    src_ref=out_ref.at[pl.ds(base1, HC)],
                    dst_ref=out_ref.at[pl.ds(base1, HC)],
                    send_sem=f1s_sems.at[e],
                    recv_sem=f1r_sems.at[e],
                    device_id=zpeer,
                    device_id_type=pl.DeviceIdType.MESH,
                )
                f1.start()
                f1rdmas[e] = f1

                zrdmas[e].wait_recv()
                base2 = p_z * Q + e * CH
                f2 = pltpu.make_async_remote_copy(
                    src_ref=out_ref.at[pl.ds(base2, HC)],
                    dst_ref=out_ref.at[pl.ds(base2, HC)],
                    send_sem=f2s_sems.at[e],
                    recv_sem=f2r_sems.at[e],
                    device_id=ypeer,
                    device_id_type=pl.DeviceIdType.MESH,
                )
                f2.start()
                f2rdmas[e] = f2

        for c in range(NC):
            xrdmas[c].wait_send()
            yrdmas[c].wait_send()
            zrdmas[c].wait_send()
            f1rdmas[c].wait_send()
            f2rdmas[c].wait_send()
            scps.pop(c).wait()
        for c in range(NC):
            f1rdmas[c].wait_recv()
            f2rdmas[c].wait_recv()

    out, _ = pl.pallas_call(
        body,
        out_shape=[
            jax.ShapeDtypeStruct((T, D), jnp.bfloat16),
            jax.ShapeDtypeStruct((Q, D), jnp.bfloat16),
        ],
        in_specs=[
            pl.BlockSpec(memory_space=pltpu.SMEM),
            pl.BlockSpec(memory_space=pl.MemorySpace.ANY),
            pl.BlockSpec(memory_space=pl.MemorySpace.ANY),
        ],
        out_specs=[
            pl.BlockSpec(memory_space=pl.MemorySpace.ANY),
            pl.BlockSpec(memory_space=pl.MemorySpace.ANY),
        ],
        scratch_shapes=[
            pltpu.VMEM((2, CH, D), jnp.float32),
            pltpu.VMEM((NC, CH, D), jnp.bfloat16),
            pltpu.VMEM((Q, 1), jnp.float32),
            pltpu.VMEM((2, CH, D), jnp.bfloat16),
            pltpu.VMEM((NC, CH, D), jnp.bfloat16),
            pltpu.SemaphoreType.DMA((NC,)),
            pltpu.SemaphoreType.DMA((NC,)),
            pltpu.SemaphoreType.DMA((NC,)),
            pltpu.SemaphoreType.DMA((NC,)),
            pltpu.SemaphoreType.DMA((NC,)),
            pltpu.SemaphoreType.DMA((NC,)),
            pltpu.SemaphoreType.DMA((NC,)),
            pltpu.SemaphoreType.DMA((NC,)),
            pltpu.SemaphoreType.DMA((NC,)),
            pltpu.SemaphoreType.DMA((NC,)),
            pltpu.SemaphoreType.DMA((2,)),
            pltpu.SemaphoreType.DMA,
            pltpu.SemaphoreType.DMA,
            pltpu.SemaphoreType.DMA((NC,)),
        ],
        compiler_params=pltpu.CompilerParams(collective_id=0),
    )(safe_ids, maskf, E)
    return out


def kernel(ids, E):
    v_loc = E.shape[0]
    my_x = lax.axis_index("x")
    local = ids - my_x * v_loc
    mask = (local >= 0) & (local < v_loc)
    safe = jnp.where(mask, local, 0).astype(jnp.int32)
    maskf = mask.astype(jnp.float32)[:, None]
    return _fused_embed_allreduce(safe, maskf, E)


# device time: 107868 ns/iter; 3.6930x vs baseline; 1.1010x over previous
import jax
import jax.numpy as jnp
from jax import lax
from jax.experimental import pallas as pl
from jax.experimental.pallas import tpu as pltpu

T = 4096
D = 2048
Q = T // 4
CH = 128
NC = Q // CH
HC = CH // 2
LAG = 2
ALAG = 4


def _fused_embed_allreduce(safe_ids, maskf, E):
    def body(ids_ref, mask_ref, e_ref, out_ref, commx_ref,
             g_v, g_bf, mask_v, b_v, s_bf,
             xs_sems, xr_sems, ys_sems, yr_sems, zs_sems, zr_sems,
             f1s_sems, f1r_sems, f2s_sems, f2r_sems,
             gsems, msem, bsem, ssems):
        my_x = lax.axis_index("x")
        my_y = lax.axis_index("y")
        my_z = lax.axis_index("z")
        zb = my_z % 2
        zpeer_z = my_z + 1 - 2 * zb
        xpeer = (1 - my_x, my_y, my_z)
        ypeer = (my_x, 1 - my_y, my_z)
        zpeer = (my_x, my_y, zpeer_z)

        p_me = 2 * my_y + zb
        p_y = 2 * (1 - my_y) + zb
        p_z = 2 * my_y + (1 - zb)
        q0 = p_me * Q

        def issue_gather(c):
            cps = []
            for i in range(CH):
                idx = ids_ref[q0 + c * CH + i]
                cp = pltpu.make_async_copy(
                    e_ref.at[pl.ds(idx, 1)],
                    g_v.at[c % 2].at[pl.ds(i, 1)],
                    gsems.at[c % 2])
                cp.start()
                cps.append(cp)
            return cps

        gather_cps = {0: issue_gather(0)}
        cpm = pltpu.make_async_copy(mask_ref.at[pl.ds(q0, Q)], mask_v, msem)
        cpm.start()

        barrier_sem = pltpu.get_barrier_semaphore()
        for nbr in (xpeer, ypeer, zpeer):
            pl.semaphore_signal(
                barrier_sem, inc=1, device_id=nbr,
                device_id_type=pl.DeviceIdType.MESH,
            )
        pl.semaphore_wait(barrier_sem, 3)
        cpm.wait()

        xrdmas = []
        yrdmas = {}
        zrdmas = {}
        f1rdmas = {}
        f2rdmas = {}
        bcps = {}
        scps = {}
        for c in range(NC + ALAG):
            d = c - LAG
            if 0 <= d < NC:
                xrdmas[d].wait_recv()
                bcp = pltpu.make_async_copy(
                    commx_ref.at[pl.ds(d * CH, CH)], b_v.at[d % 2], bsem)
                bcp.start()
                bcps[d] = bcp

            if c < NC:
                if c + 1 < NC:
                    gather_cps[c + 1] = issue_gather(c + 1)
                for cp in gather_cps.pop(c):
                    cp.wait()
                g_bf[c] = (
                    g_v[c % 2] * mask_v[pl.ds(c * CH, CH)]
                ).astype(jnp.bfloat16)
                rdma = pltpu.make_async_remote_copy(
                    src_ref=g_bf.at[c],
                    dst_ref=commx_ref.at[pl.ds(c * CH, CH)],
                    send_sem=xs_sems.at[c],
                    recv_sem=xr_sems.at[c],
                    device_id=xpeer,
                    device_id_type=pl.DeviceIdType.MESH,
                )
                rdma.start()
                xrdmas.append(rdma)

            if 0 <= d < NC:
                bcps.pop(d).wait()
                rows = pl.ds(q0 + d * CH, CH)
                s_bf[d] = (
                    g_bf[d].astype(jnp.float32)
                    + b_v[d % 2].astype(jnp.float32)
                ).astype(jnp.bfloat16)
                for peer, ss, rr, book in (
                    (ypeer, ys_sems, yr_sems, yrdmas),
                    (zpeer, zs_sems, zr_sems, zrdmas),
                ):
                    prdma = pltpu.make_async_remote_copy(
                        src_ref=s_bf.at[d],
                        dst_ref=out_ref.at[rows],
                        send_sem=ss.at[d],
                        recv_sem=rr.at[d],
                        device_id=peer,
                        device_id_type=pl.DeviceIdType.MESH,
                    )
                    prdma.start()
                    book[d] = prdma
                scp = pltpu.make_async_copy(
                    s_bf.at[d], out_ref.at[rows], ssems.at[d])
                scp.start()
                scps[d] = scp

            e = c - ALAG
            if 0 <= e < NC:
                yrdmas[e].wait_recv()
                base1 = p_y * Q + e * CH + HC
                f1 = pltpu.make_async_remote_copy(
                    src_ref=out_ref.at[pl.ds(base1, HC)],
                    dst_ref=out_ref.at[pl.ds(base1, HC)],
                    send_sem=f1s_sems.at[e],
                    recv_sem=f1r_sems.at[e],
                    device_id=zpeer,
                    device_id_type=pl.DeviceIdType.MESH,
                )
                f1.start()
                f1rdmas[e] = f1

                zrdmas[e].wait_recv()
                base2 = p_z * Q + e * CH
                f2 = pltpu.make_async_remote_copy(
                    src_ref=out_ref.at[pl.ds(base2, HC)],
                    dst_ref=out_ref.at[pl.ds(base2, HC)],
                    send_sem=f2s_sems.at[e],
                    recv_sem=f2r_sems.at[e],
                    device_id=ypeer,
                    device_id_type=pl.DeviceIdType.MESH,
                )
                f2.start()
                f2rdmas[e] = f2

        for c in range(NC):
            xrdmas[c].wait_send()
            yrdmas[c].wait_send()
            zrdmas[c].wait_send()
            f1rdmas[c].wait_send()
            f2rdmas[c].wait_send()
            scps.pop(c).wait()
        for c in range(NC):
            f1rdmas[c].wait_recv()
            f2rdmas[c].wait_recv()

    out, _ = pl.pallas_call(
        body,
        out_shape=[
            jax.ShapeDtypeStruct((T, D), jnp.bfloat16),
            jax.ShapeDtypeStruct((Q, D), jnp.bfloat16),
        ],
        in_specs=[
            pl.BlockSpec(memory_space=pltpu.SMEM),
            pl.BlockSpec(memory_space=pl.MemorySpace.ANY),
            pl.BlockSpec(memory_space=pl.MemorySpace.ANY),
        ],
        out_specs=[
            pl.BlockSpec(memory_space=pl.MemorySpace.ANY),
            pl.BlockSpec(memory_space=pl.MemorySpace.ANY),
        ],
        scratch_shapes=[
            pltpu.VMEM((2, CH, D), jnp.float32),
            pltpu.VMEM((NC, CH, D), jnp.bfloat16),
            pltpu.VMEM((Q, 1), jnp.float32),
            pltpu.VMEM((2, CH, D), jnp.bfloat16),
            pltpu.VMEM((NC, CH, D), jnp.bfloat16),
            pltpu.SemaphoreType.DMA((NC,)),
            pltpu.SemaphoreType.DMA((NC,)),
            pltpu.SemaphoreType.DMA((NC,)),
            pltpu.SemaphoreType.DMA((NC,)),
            pltpu.SemaphoreType.DMA((NC,)),
            pltpu.SemaphoreType.DMA((NC,)),
            pltpu.SemaphoreType.DMA((NC,)),
            pltpu.SemaphoreType.DMA((NC,)),
            pltpu.SemaphoreType.DMA((NC,)),
            pltpu.SemaphoreType.DMA((NC,)),
            pltpu.SemaphoreType.DMA((2,)),
            pltpu.SemaphoreType.DMA,
            pltpu.SemaphoreType.DMA,
            pltpu.SemaphoreType.DMA((NC,)),
        ],
        compiler_params=pltpu.CompilerParams(collective_id=0),
    )(safe_ids, maskf, E)
    return out


def kernel(ids, E):
    v_loc = E.shape[0]
    my_x = lax.axis_index("x")
    local = ids - my_x * v_loc
    mask = (local >= 0) & (local < v_loc)
    safe = jnp.where(mask, local, 0).astype(jnp.int32)
    maskf = mask.astype(jnp.float32)[:, None]
    return _fused_embed_allreduce(safe, maskf, E)


# device time: 105184 ns/iter; 3.7872x vs baseline; 1.0255x over previous
import jax
import jax.numpy as jnp
from jax import lax
from jax.experimental import pallas as pl
from jax.experimental.pallas import tpu as pltpu

T = 4096
D = 2048
Q = T // 4
CH = 128
NC = Q // CH
HC = CH // 2
LAG = 2
ALAG = 4


def _fused_embed_allreduce(safe_ids, mask_i32, maskf, E):
    def body(ids_ref, mski_ref, mask_ref, e_ref, out_ref, commx_ref,
             g_v, g_bf, mask_v, b_v, s_bf,
             xs_sems, xr_sems, ys_sems, yr_sems, zs_sems, zr_sems,
             f1s_sems, f1r_sems, f2s_sems, f2r_sems,
             gsems, msem, bsem, ssems):
        my_x = lax.axis_index("x")
        my_y = lax.axis_index("y")
        my_z = lax.axis_index("z")
        zb = my_z % 2
        zpeer_z = my_z + 1 - 2 * zb
        xpeer = (1 - my_x, my_y, my_z)
        ypeer = (my_x, 1 - my_y, my_z)
        zpeer = (my_x, my_y, zpeer_z)

        p_me = 2 * my_y + zb
        p_y = 2 * (1 - my_y) + zb
        p_z = 2 * my_y + (1 - zb)
        q0 = p_me * Q

        def issue_gather(c):
            cps = []
            for i in range(CH):
                pos = q0 + c * CH + i
                idx = ids_ref[pos]
                owned = mski_ref[pos] != 0
                cp = pltpu.make_async_copy(
                    e_ref.at[pl.ds(idx, 1)],
                    g_v.at[c % 2].at[pl.ds(i, 1)],
                    gsems.at[c % 2])

                @pl.when(owned)
                def _(cp=cp):
                    cp.start()

                cps.append((cp, owned))
            return cps

        def wait_gather(cps):
            for cp, owned in cps:
                @pl.when(owned)
                def _(cp=cp):
                    cp.wait()

        gather_cps = {0: issue_gather(0)}
        cpm = pltpu.make_async_copy(mask_ref.at[pl.ds(q0, Q)], mask_v, msem)
        cpm.start()

        barrier_sem = pltpu.get_barrier_semaphore()
        for nbr in (xpeer, ypeer, zpeer):
            pl.semaphore_signal(
                barrier_sem, inc=1, device_id=nbr,
                device_id_type=pl.DeviceIdType.MESH,
            )
        pl.semaphore_wait(barrier_sem, 3)
        cpm.wait()

        xrdmas = []
        yrdmas = {}
        zrdmas = {}
        f1rdmas = {}
        f2rdmas = {}
        bcps = {}
        scps = {}
        for c in range(NC + ALAG):
            d = c - LAG
            if 0 <= d < NC:
                xrdmas[d].wait_recv()
                bcp = pltpu.make_async_copy(
                    commx_ref.at[pl.ds(d * CH, CH)], b_v.at[d % 2], bsem)
                bcp.start()
                bcps[d] = bcp

            if c < NC:
                if c + 1 < NC:
                    gather_cps[c + 1] = issue_gather(c + 1)
                wait_gather(gather_cps.pop(c))
                g_bf[c] = (
                    g_v[c % 2] * mask_v[pl.ds(c * CH, CH)]
                ).astype(jnp.bfloat16)
                rdma = pltpu.make_async_remote_copy(
                    src_ref=g_bf.at[c],
                    dst_ref=commx_ref.at[pl.ds(c * CH, CH)],
                    send_sem=xs_sems.at[c],
                    recv_sem=xr_sems.at[c],
                    device_id=xpeer,
                    device_id_type=pl.DeviceIdType.MESH,
                )
                rdma.start()
                xrdmas.append(rdma)

            if 0 <= d < NC:
                bcps.pop(d).wait()
                rows = pl.ds(q0 + d * CH, CH)
                s_bf[d] = (
                    g_bf[d].astype(jnp.float32)
                    + b_v[d % 2].astype(jnp.float32)
                ).astype(jnp.bfloat16)
                for peer, ss, rr, book in (
                    (ypeer, ys_sems, yr_sems, yrdmas),
                    (zpeer, zs_sems, zr_sems, zrdmas),
                ):
                    prdma = pltpu.make_async_remote_copy(
                        src_ref=s_bf.at[d],
                        dst_ref=out_ref.at[rows],
                        send_sem=ss.at[d],
                        recv_sem=rr.at[d],
                        device_id=peer,
                        device_id_type=pl.DeviceIdType.MESH,
                    )
                    prdma.start()
                    book[d] = prdma
                scp = pltpu.make_async_copy(
                    s_bf.at[d], out_ref.at[rows], ssems.at[d])
                scp.start()
                scps[d] = scp

            e = c - ALAG
            if 0 <= e < NC:
                yrdmas[e].wait_recv()
                base1 = p_y * Q + e * CH + HC
                f1 = pltpu.make_async_remote_copy(
                    src_ref=out_ref.at[pl.ds(base1, HC)],
                    dst_ref=out_ref.at[pl.ds(base1, HC)],
                    send_sem=f1s_sems.at[e],
                    recv_sem=f1r_sems.at[e],
                    device_id=zpeer,
                    device_id_type=pl.DeviceIdType.MESH,
                )
                f1.start()
                f1rdmas[e] = f1

                zrdmas[e].wait_recv()
                base2 = p_z * Q + e * CH
                f2 = pltpu.make_async_remote_copy(
                    src_ref=out_ref.at[pl.ds(base2, HC)],
                    dst_ref=out_ref.at[pl.ds(base2, HC)],
                    send_sem=f2s_sems.at[e],
                    recv_sem=f2r_sems.at[e],
                    device_id=ypeer,
                    device_id_type=pl.DeviceIdType.MESH,
                )
                f2.start()
                f2rdmas[e] = f2

        for c in range(NC):
            xrdmas[c].wait_send()
            yrdmas[c].wait_send()
            zrdmas[c].wait_send()
            f1rdmas[c].wait_send()
            f2rdmas[c].wait_send()
            scps.pop(c).wait()
        for c in range(NC):
            f1rdmas[c].wait_recv()
            f2rdmas[c].wait_recv()

    out, _ = pl.pallas_call(
        body,
        out_shape=[
            jax.ShapeDtypeStruct((T, D), jnp.bfloat16),
            jax.ShapeDtypeStruct((Q, D), jnp.bfloat16),
        ],
        in_specs=[
            pl.BlockSpec(memory_space=pltpu.SMEM),
            pl.BlockSpec(memory_space=pltpu.SMEM),
            pl.BlockSpec(memory_space=pl.MemorySpace.ANY),
            pl.BlockSpec(memory_space=pl.MemorySpace.ANY),
        ],
        out_specs=[
            pl.BlockSpec(memory_space=pl.MemorySpace.ANY),
            pl.BlockSpec(memory_space=pl.MemorySpace.ANY),
        ],
        scratch_shapes=[
            pltpu.VMEM((2, CH, D), jnp.float32),
            pltpu.VMEM((NC, CH, D), jnp.bfloat16),
            pltpu.VMEM((Q, 1), jnp.float32),
            pltpu.VMEM((2, CH, D), jnp.bfloat16),
            pltpu.VMEM((NC, CH, D), jnp.bfloat16),
            pltpu.SemaphoreType.DMA((NC,)),
            pltpu.SemaphoreType.DMA((NC,)),
            pltpu.SemaphoreType.DMA((NC,)),
            pltpu.SemaphoreType.DMA((NC,)),
            pltpu.SemaphoreType.DMA((NC,)),
            pltpu.SemaphoreType.DMA((NC,)),
            pltpu.SemaphoreType.DMA((NC,)),
            pltpu.SemaphoreType.DMA((NC,)),
            pltpu.SemaphoreType.DMA((NC,)),
            pltpu.SemaphoreType.DMA((NC,)),
            pltpu.SemaphoreType.DMA((2,)),
            pltpu.SemaphoreType.DMA,
            pltpu.SemaphoreType.DMA,
            pltpu.SemaphoreType.DMA((NC,)),
        ],
        compiler_params=pltpu.CompilerParams(collective_id=0),
    )(safe_ids, mask_i32, maskf, E)
    return out


def kernel(ids, E):
    v_loc = E.shape[0]
    my_x = lax.axis_index("x")
    local = ids - my_x * v_loc
    mask = (local >= 0) & (local < v_loc)
    safe = jnp.where(mask, local, 0).astype(jnp.int32)
    mask_i32 = mask.astype(jnp.int32)
    maskf = mask.astype(jnp.float32)[:, None]
    return _fused_embed_allreduce(safe, mask_i32, maskf, E)


# device time: 101962 ns/iter; 3.9069x vs baseline; 1.0316x over previous
import jax
import jax.numpy as jnp
from jax import lax
from jax.experimental import pallas as pl
from jax.experimental.pallas import tpu as pltpu

T = 4096
D = 2048
Q = T // 4
SIZES = (64, 64, 128, 128, 128, 128, 128, 128, 128)
OFFS = tuple(sum(SIZES[:i]) for i in range(len(SIZES)))
NC = len(SIZES)
CHMAX = max(SIZES)
LAG = 2
ALAG = 3


def _fused_embed_allreduce(safe_ids, mask_i32, maskf, E):
    def body(ids_ref, mski_ref, mask_ref, e_ref, out_ref, commx_ref,
             g_v, g_bf, mask_v, b_v, s_bf,
             xs_sems, xr_sems, ys_sems, yr_sems, zs_sems, zr_sems,
             f1s_sems, f1r_sems, f2s_sems, f2r_sems,
             gsems, msem, bsem, ssems):
        my_x = lax.axis_index("x")
        my_y = lax.axis_index("y")
        my_z = lax.axis_index("z")
        zb = my_z % 2
        zpeer_z = my_z + 1 - 2 * zb
        xpeer = (1 - my_x, my_y, my_z)
        ypeer = (my_x, 1 - my_y, my_z)
        zpeer = (my_x, my_y, zpeer_z)

        p_me = 2 * my_y + zb
        p_y = 2 * (1 - my_y) + zb
        p_z = 2 * my_y + (1 - zb)
        q0 = p_me * Q

        def issue_gather(c):
            cps = []
            for i in range(SIZES[c]):
                pos = q0 + OFFS[c] + i
                idx = ids_ref[pos]
                owned = mski_ref[pos] != 0
                cp = pltpu.make_async_copy(
                    e_ref.at[pl.ds(idx, 1)],
                    g_v.at[c % 2].at[pl.ds(i, 1)],
                    gsems.at[c % 2])

                @pl.when(owned)
                def _(cp=cp):
                    cp.start()

                cps.append((cp, owned))
            return cps

        def wait_gather(cps):
            for cp, owned in cps:
                @pl.when(owned)
                def _(cp=cp):
                    cp.wait()

        gather_cps = {0: issue_gather(0)}
        cpm = pltpu.make_async_copy(mask_ref.at[pl.ds(q0, Q)], mask_v, msem)
        cpm.start()

        barrier_sem = pltpu.get_barrier_semaphore()
        for nbr in (xpeer, ypeer, zpeer):
            pl.semaphore_signal(
                barrier_sem, inc=1, device_id=nbr,
                device_id_type=pl.DeviceIdType.MESH,
            )
        pl.semaphore_wait(barrier_sem, 3)
        cpm.wait()

        xrdmas = []
        yrdmas = {}
        zrdmas = {}
        f1rdmas = {}
        f2rdmas = {}
        bcps = {}
        scps = {}
        for c in range(NC + ALAG):
            d = c - LAG
            if 0 <= d < NC:
                xrdmas[d].wait_recv()
                bcp = pltpu.make_async_copy(
                    commx_ref.at[pl.ds(OFFS[d], SIZES[d])],
                    b_v.at[d % 2].at[pl.ds(0, SIZES[d])], bsem)
                bcp.start()
                bcps[d] = bcp

            if c < NC:
                if c + 1 < NC:
                    gather_cps[c + 1] = issue_gather(c + 1)
                wait_gather(gather_cps.pop(c))
                g_bf[c] = (
                    g_v[c % 2] * mask_v[pl.ds(OFFS[c], CHMAX)]
                ).astype(jnp.bfloat16)
                rdma = pltpu.make_async_remote_copy(
                    src_ref=g_bf.at[c].at[pl.ds(0, SIZES[c])],
                    dst_ref=commx_ref.at[pl.ds(OFFS[c], SIZES[c])],
                    send_sem=xs_sems.at[c],
                    recv_sem=xr_sems.at[c],
                    device_id=xpeer,
                    device_id_type=pl.DeviceIdType.MESH,
                )
                rdma.start()
                xrdmas.append(rdma)

            if 0 <= d < NC:
                bcps.pop(d).wait()
                rows = pl.ds(q0 + OFFS[d], SIZES[d])
                s_bf[d] = (
                    g_bf[d].astype(jnp.float32)
                    + b_v[d % 2].astype(jnp.float32)
                ).astype(jnp.bfloat16)
                for peer, ss, rr, book in (
                    (ypeer, ys_sems, yr_sems, yrdmas),
                    (zpeer, zs_sems, zr_sems, zrdmas),
                ):
                    prdma = pltpu.make_async_remote_copy(
                        src_ref=s_bf.at[d].at[pl.ds(0, SIZES[d])],
                        dst_ref=out_ref.at[rows],
                        send_sem=ss.at[d],
                        recv_sem=rr.at[d],
                        device_id=peer,
                        device_id_type=pl.DeviceIdType.MESH,
                    )
                    prdma.start()
                    book[d] = prdma
                scp = pltpu.make_async_copy(
                    s_bf.at[d].at[pl.ds(0, SIZES[d])], out_ref.at[rows],
                    ssems.at[d])
                scp.start()
                scps[d] = scp

            e = c - ALAG
            if 0 <= e < NC:
                fs = SIZES[e] // 2
                yrdmas[e].wait_recv()
                base1 = p_y * Q + OFFS[e] + fs
                f1 = pltpu.make_async_remote_copy(
                    src_ref=out_ref.at[pl.ds(base1, fs)],
                    dst_ref=out_ref.at[pl.ds(base1, fs)],
                    send_sem=f1s_sems.at[e],
                    recv_sem=f1r_sems.at[e],
                    device_id=zpeer,
                    device_id_type=pl.DeviceIdType.MESH,
                )
                f1.start()
                f1rdmas[e] = f1

                zrdmas[e].wait_recv()
                base2 = p_z * Q + OFFS[e]
                f2 = pltpu.make_async_remote_copy(
                    src_ref=out_ref.at[pl.ds(base2, fs)],
                    dst_ref=out_ref.at[pl.ds(base2, fs)],
                    send_sem=f2s_sems.at[e],
                    recv_sem=f2r_sems.at[e],
                    device_id=ypeer,
                    device_id_type=pl.DeviceIdType.MESH,
                )
                f2.start()
                f2rdmas[e] = f2

        for c in range(NC):
            xrdmas[c].wait_send()
            yrdmas[c].wait_send()
            zrdmas[c].wait_send()
            f1rdmas[c].wait_send()
            f2rdmas[c].wait_send()
            scps.pop(c).wait()
        for c in range(NC):
            f1rdmas[c].wait_recv()
            f2rdmas[c].wait_recv()

    out, _ = pl.pallas_call(
        body,
        out_shape=[
            jax.ShapeDtypeStruct((T, D), jnp.bfloat16),
            jax.ShapeDtypeStruct((Q, D), jnp.bfloat16),
        ],
        in_specs=[
            pl.BlockSpec(memory_space=pltpu.SMEM),
            pl.BlockSpec(memory_space=pltpu.SMEM),
            pl.BlockSpec(memory_space=pl.MemorySpace.ANY),
            pl.BlockSpec(memory_space=pl.MemorySpace.ANY),
        ],
        out_specs=[
            pl.BlockSpec(memory_space=pl.MemorySpace.ANY),
            pl.BlockSpec(memory_space=pl.MemorySpace.ANY),
        ],
        scratch_shapes=[
            pltpu.VMEM((2, CHMAX, D), jnp.float32),
            pltpu.VMEM((NC, CHMAX, D), jnp.bfloat16),
            pltpu.VMEM((Q, 1), jnp.float32),
            pltpu.VMEM((2, CHMAX, D), jnp.bfloat16),
            pltpu.VMEM((NC, CHMAX, D), jnp.bfloat16),
            pltpu.SemaphoreType.DMA((NC,)),
            pltpu.SemaphoreType.DMA((NC,)),
            pltpu.SemaphoreType.DMA((NC,)),
            pltpu.SemaphoreType.DMA((NC,)),
            pltpu.SemaphoreType.DMA((NC,)),
            pltpu.SemaphoreType.DMA((NC,)),
            pltpu.SemaphoreType.DMA((NC,)),
            pltpu.SemaphoreType.DMA((NC,)),
            pltpu.SemaphoreType.DMA((NC,)),
            pltpu.SemaphoreType.DMA((NC,)),
            pltpu.SemaphoreType.DMA((2,)),
            pltpu.SemaphoreType.DMA,
            pltpu.SemaphoreType.DMA,
            pltpu.SemaphoreType.DMA((NC,)),
        ],
        compiler_params=pltpu.CompilerParams(collective_id=0),
    )(safe_ids, mask_i32, maskf, E)
    return out


def kernel(ids, E):
    v_loc = E.shape[0]
    my_x = lax.axis_index("x")
    local = ids - my_x * v_loc
    mask = (local >= 0) & (local < v_loc)
    safe = jnp.where(mask, local, 0).astype(jnp.int32)
    mask_i32 = mask.astype(jnp.int32)
    maskf = mask.astype(jnp.float32)[:, None]
    return _fused_embed_allreduce(safe, mask_i32, maskf, E)


# device time: 90500 ns/iter; 4.4017x vs baseline; 1.1267x over previous
import jax
import jax.numpy as jnp
from jax import lax
from jax.experimental import pallas as pl
from jax.experimental.pallas import tpu as pltpu

T = 4096
D = 2048
Q = T // 4
SIZES = (64, 64, 128, 128, 128, 128, 128, 128, 128)
OFFS = tuple(sum(SIZES[:i]) for i in range(len(SIZES)))
NC = len(SIZES)
CHMAX = max(SIZES)
LAG = 2
ALAG = 3


def _fused_embed_allreduce(safe_ids, mask_i32, maskf, E):
    def body(ids_ref, mski_ref, mask_ref, e_ref, out_ref,
             g_v, g_bf, mask_v, commx_v, s_bf,
             xs_sems, xr_sems, ys_sems, yr_sems, zs_sems, zr_sems,
             f1s_sems, f1r_sems, f2s_sems, f2r_sems,
             gsems, msem, ssems):
        my_x = lax.axis_index("x")
        my_y = lax.axis_index("y")
        my_z = lax.axis_index("z")
        zb = my_z % 2
        zpeer_z = my_z + 1 - 2 * zb
        xpeer = (1 - my_x, my_y, my_z)
        ypeer = (my_x, 1 - my_y, my_z)
        zpeer = (my_x, my_y, zpeer_z)

        p_me = 2 * my_y + zb
        p_y = 2 * (1 - my_y) + zb
        p_z = 2 * my_y + (1 - zb)
        q0 = p_me * Q

        def issue_gather(c):
            cps = []
            for i in range(SIZES[c]):
                pos = q0 + OFFS[c] + i
                idx = ids_ref[pos]
                owned = mski_ref[pos] != 0
                cp = pltpu.make_async_copy(
                    e_ref.at[pl.ds(idx, 1)],
                    g_v.at[c % 2].at[pl.ds(i, 1)],
                    gsems.at[c % 2])

                @pl.when(owned)
                def _(cp=cp):
                    cp.start()

                cps.append((cp, owned))
            return cps

        def wait_gather(cps):
            for cp, owned in cps:
                @pl.when(owned)
                def _(cp=cp):
                    cp.wait()

        gather_cps = {0: issue_gather(0)}
        cpm = pltpu.make_async_copy(mask_ref.at[pl.ds(q0, Q)], mask_v, msem)
        cpm.start()

        barrier_sem = pltpu.get_barrier_semaphore()
        for nbr in (xpeer, ypeer, zpeer):
            pl.semaphore_signal(
                barrier_sem, inc=1, device_id=nbr,
                device_id_type=pl.DeviceIdType.MESH,
            )
        pl.semaphore_wait(barrier_sem, 3)
        cpm.wait()

        xrdmas = []
        yrdmas = {}
        zrdmas = {}
        f1rdmas = {}
        f2rdmas = {}
        scps = {}
        for c in range(NC + ALAG):
            d = c - LAG
            if c < NC:
                if c + 1 < NC:
                    gather_cps[c + 1] = issue_gather(c + 1)
                wait_gather(gather_cps.pop(c))
                g_bf[c] = (
                    g_v[c % 2] * mask_v[pl.ds(OFFS[c], CHMAX)]
                ).astype(jnp.bfloat16)
                rdma = pltpu.make_async_remote_copy(
                    src_ref=g_bf.at[c].at[pl.ds(0, SIZES[c])],
                    dst_ref=commx_v.at[pl.ds(OFFS[c], SIZES[c])],
                    send_sem=xs_sems.at[c],
                    recv_sem=xr_sems.at[c],
                    device_id=xpeer,
                    device_id_type=pl.DeviceIdType.MESH,
                )
                rdma.start()
                xrdmas.append(rdma)

            if 0 <= d < NC:
                xrdmas[d].wait_recv()
                rows = pl.ds(q0 + OFFS[d], SIZES[d])
                s_bf[d] = (
                    g_bf[d].astype(jnp.float32)
                    + commx_v[pl.ds(OFFS[d], CHMAX)].astype(jnp.float32)
                ).astype(jnp.bfloat16)
                for peer, ss, rr, book in (
                    (ypeer, ys_sems, yr_sems, yrdmas),
                    (zpeer, zs_sems, zr_sems, zrdmas),
                ):
                    prdma = pltpu.make_async_remote_copy(
                        src_ref=s_bf.at[d].at[pl.ds(0, SIZES[d])],
                        dst_ref=out_ref.at[rows],
                        send_sem=ss.at[d],
                        recv_sem=rr.at[d],
                        device_id=peer,
                        device_id_type=pl.DeviceIdType.MESH,
                    )
                    prdma.start()
                    book[d] = prdma
                scp = pltpu.make_async_copy(
                    s_bf.at[d].at[pl.ds(0, SIZES[d])], out_ref.at[rows],
                    ssems.at[d])
                scp.start()
                scps[d] = scp

            e = c - ALAG
            if 0 <= e < NC:
                fs = SIZES[e] // 2
                yrdmas[e].wait_recv()
                base1 = p_y * Q + OFFS[e] + fs
                f1 = pltpu.make_async_remote_copy(
                    src_ref=out_ref.at[pl.ds(base1, fs)],
                    dst_ref=out_ref.at[pl.ds(base1, fs)],
                    send_sem=f1s_sems.at[e],
                    recv_sem=f1r_sems.at[e],
                    device_id=zpeer,
                    device_id_type=pl.DeviceIdType.MESH,
                )
                f1.start()
                f1rdmas[e] = f1

                zrdmas[e].wait_recv()
                base2 = p_z * Q + OFFS[e]
                f2 = pltpu.make_async_remote_copy(
                    src_ref=out_ref.at[pl.ds(base2, fs)],
                    dst_ref=out_ref.at[pl.ds(base2, fs)],
                    send_sem=f2s_sems.at[e],
                    recv_sem=f2r_sems.at[e],
                    device_id=ypeer,
                    device_id_type=pl.DeviceIdType.MESH,
                )
                f2.start()
                f2rdmas[e] = f2

        for c in range(NC):
            xrdmas[c].wait_send()
            yrdmas[c].wait_send()
            zrdmas[c].wait_send()
            f1rdmas[c].wait_send()
            f2rdmas[c].wait_send()
            scps.pop(c).wait()
        for c in range(NC):
            f1rdmas[c].wait_recv()
            f2rdmas[c].wait_recv()

    out = pl.pallas_call(
        body,
        out_shape=jax.ShapeDtypeStruct((T, D), jnp.bfloat16),
        in_specs=[
            pl.BlockSpec(memory_space=pltpu.SMEM),
            pl.BlockSpec(memory_space=pltpu.SMEM),
            pl.BlockSpec(memory_space=pl.MemorySpace.ANY),
            pl.BlockSpec(memory_space=pl.MemorySpace.ANY),
        ],
        out_specs=pl.BlockSpec(memory_space=pl.MemorySpace.ANY),
        scratch_shapes=[
            pltpu.VMEM((2, CHMAX, D), jnp.float32),
            pltpu.VMEM((NC, CHMAX, D), jnp.bfloat16),
            pltpu.VMEM((Q, 1), jnp.float32),
            pltpu.VMEM((Q, D), jnp.bfloat16),
            pltpu.VMEM((NC, CHMAX, D), jnp.bfloat16),
            pltpu.SemaphoreType.DMA((NC,)),
            pltpu.SemaphoreType.DMA((NC,)),
            pltpu.SemaphoreType.DMA((NC,)),
            pltpu.SemaphoreType.DMA((NC,)),
            pltpu.SemaphoreType.DMA((NC,)),
            pltpu.SemaphoreType.DMA((NC,)),
            pltpu.SemaphoreType.DMA((NC,)),
            pltpu.SemaphoreType.DMA((NC,)),
            pltpu.SemaphoreType.DMA((NC,)),
            pltpu.SemaphoreType.DMA((NC,)),
            pltpu.SemaphoreType.DMA((2,)),
            pltpu.SemaphoreType.DMA,
            pltpu.SemaphoreType.DMA((NC,)),
        ],
        compiler_params=pltpu.CompilerParams(collective_id=0),
    )(safe_ids, mask_i32, maskf, E)
    return out


def kernel(ids, E):
    v_loc = E.shape[0]
    my_x = lax.axis_index("x")
    local = ids - my_x * v_loc
    mask = (local >= 0) & (local < v_loc)
    safe = jnp.where(mask, local, 0).astype(jnp.int32)
    mask_i32 = mask.astype(jnp.int32)
    maskf = mask.astype(jnp.float32)[:, None]
    return _fused_embed_allreduce(safe, mask_i32, maskf, E)


# device time: 90362 ns/iter; 4.4084x vs baseline; 1.0015x over previous
import jax
import jax.numpy as jnp
from jax import lax
from jax.experimental import pallas as pl
from jax.experimental.pallas import tpu as pltpu

T = 4096
D = 2048
Q = T // 4
SIZES = (32, 32, 64, 128, 128, 128, 128, 128, 128, 128)
OFFS = tuple(sum(SIZES[:i]) for i in range(len(SIZES)))
NC = len(SIZES)
CHMAX = max(SIZES)
LAG = 2
ALAG = 3


def _fused_embed_allreduce(safe_ids, mask_i32, maskf, E):
    def body(ids_ref, mski_ref, mask_ref, e_ref, out_ref,
             g_v, g_bf, mask_v, commx_v, s_bf,
             xs_sems, xr_sems, ys_sems, yr_sems, zs_sems, zr_sems,
             f1s_sems, f1r_sems, f2s_sems, f2r_sems,
             gsems, msem, ssems):
        my_x = lax.axis_index("x")
        my_y = lax.axis_index("y")
        my_z = lax.axis_index("z")
        zb = my_z % 2
        zpeer_z = my_z + 1 - 2 * zb
        xpeer = (1 - my_x, my_y, my_z)
        ypeer = (my_x, 1 - my_y, my_z)
        zpeer = (my_x, my_y, zpeer_z)

        p_me = 2 * my_y + zb
        p_y = 2 * (1 - my_y) + zb
        p_z = 2 * my_y + (1 - zb)
        q0 = p_me * Q

        def issue_gather(c):
            cps = []
            for i in range(SIZES[c]):
                pos = q0 + OFFS[c] + i
                idx = ids_ref[pos]
                owned = mski_ref[pos] != 0
                cp = pltpu.make_async_copy(
                    e_ref.at[pl.ds(idx, 1)],
                    g_v.at[c % 2].at[pl.ds(i, 1)],
                    gsems.at[c % 2])

                @pl.when(owned)
                def _(cp=cp):
                    cp.start()

                cps.append((cp, owned))
            return cps

        def wait_gather(cps):
            for cp, owned in cps:
                @pl.when(owned)
                def _(cp=cp):
                    cp.wait()

        gather_cps = {0: issue_gather(0)}
        cpm = pltpu.make_async_copy(mask_ref.at[pl.ds(q0, Q)], mask_v, msem)
        cpm.start()

        barrier_sem = pltpu.get_barrier_semaphore()
        for nbr in (xpeer, ypeer, zpeer):
            pl.semaphore_signal(
                barrier_sem, inc=1, device_id=nbr,
                device_id_type=pl.DeviceIdType.MESH,
            )
        pl.semaphore_wait(barrier_sem, 3)
        cpm.wait()

        xrdmas = []
        yrdmas = {}
        zrdmas = {}
        f1rdmas = {}
        f2rdmas = {}
        scps = {}
        for c in range(NC + ALAG):
            d = c - LAG
            if c < NC:
                if c + 1 < NC:
                    gather_cps[c + 1] = issue_gather(c + 1)
                wait_gather(gather_cps.pop(c))
                g_bf[c] = (
                    g_v[c % 2] * mask_v[pl.ds(OFFS[c], CHMAX)]
                ).astype(jnp.bfloat16)
                rdma = pltpu.make_async_remote_copy(
                    src_ref=g_bf.at[c].at[pl.ds(0, SIZES[c])],
                    dst_ref=commx_v.at[pl.ds(OFFS[c], SIZES[c])],
                    send_sem=xs_sems.at[c],
                    recv_sem=xr_sems.at[c],
                    device_id=xpeer,
                    device_id_type=pl.DeviceIdType.MESH,
                )
                rdma.start()
                xrdmas.append(rdma)

            if 0 <= d < NC:
                xrdmas[d].wait_recv()
                rows = pl.ds(q0 + OFFS[d], SIZES[d])
                s_bf[d] = (
                    g_bf[d].astype(jnp.float32)
                    + commx_v[pl.ds(OFFS[d], CHMAX)].astype(jnp.float32)
                ).astype(jnp.bfloat16)
                for peer, ss, rr, book in (
                    (ypeer, ys_sems, yr_sems, yrdmas),
                    (zpeer, zs_sems, zr_sems, zrdmas),
                ):
                    prdma = pltpu.make_async_remote_copy(
                        src_ref=s_bf.at[d].at[pl.ds(0, SIZES[d])],
                        dst_ref=out_ref.at[rows],
                        send_sem=ss.at[d],
                        recv_sem=rr.at[d],
                        device_id=peer,
                        device_id_type=pl.DeviceIdType.MESH,
                    )
                    prdma.start()
                    book[d] = prdma
                scp = pltpu.make_async_copy(
                    s_bf.at[d].at[pl.ds(0, SIZES[d])], out_ref.at[rows],
                    ssems.at[d])
                scp.start()
                scps[d] = scp

            e = c - ALAG
            if 0 <= e < NC:
                fs = SIZES[e] // 2
                yrdmas[e].wait_recv()
                base1 = p_y * Q + OFFS[e] + fs
                f1 = pltpu.make_async_remote_copy(
                    src_ref=out_ref.at[pl.ds(base1, fs)],
                    dst_ref=out_ref.at[pl.ds(base1, fs)],
                    send_sem=f1s_sems.at[e],
                    recv_sem=f1r_sems.at[e],
                    device_id=zpeer,
                    device_id_type=pl.DeviceIdType.MESH,
                )
                f1.start()
                f1rdmas[e] = f1

                zrdmas[e].wait_recv()
                base2 = p_z * Q + OFFS[e]
                f2 = pltpu.make_async_remote_copy(
                    src_ref=out_ref.at[pl.ds(base2, fs)],
                    dst_ref=out_ref.at[pl.ds(base2, fs)],
                    send_sem=f2s_sems.at[e],
                    recv_sem=f2r_sems.at[e],
                    device_id=ypeer,
                    device_id_type=pl.DeviceIdType.MESH,
                )
                f2.start()
                f2rdmas[e] = f2

        for c in range(NC):
            xrdmas[c].wait_send()
            yrdmas[c].wait_send()
            zrdmas[c].wait_send()
            f1rdmas[c].wait_send()
            f2rdmas[c].wait_send()
            scps.pop(c).wait()
        for c in range(NC):
            f1rdmas[c].wait_recv()
            f2rdmas[c].wait_recv()

    out = pl.pallas_call(
        body,
        out_shape=jax.ShapeDtypeStruct((T, D), jnp.bfloat16),
        in_specs=[
            pl.BlockSpec(memory_space=pltpu.SMEM),
            pl.BlockSpec(memory_space=pltpu.SMEM),
            pl.BlockSpec(memory_space=pl.MemorySpace.ANY),
            pl.BlockSpec(memory_space=pl.MemorySpace.ANY),
        ],
        out_specs=pl.BlockSpec(memory_space=pl.MemorySpace.ANY),
        scratch_shapes=[
            pltpu.VMEM((2, CHMAX, D), jnp.float32),
            pltpu.VMEM((NC, CHMAX, D), jnp.bfloat16),
            pltpu.VMEM((Q, 1), jnp.float32),
            pltpu.VMEM((Q, D), jnp.bfloat16),
            pltpu.VMEM((NC, CHMAX, D), jnp.bfloat16),
            pltpu.SemaphoreType.DMA((NC,)),
            pltpu.SemaphoreType.DMA((NC,)),
            pltpu.SemaphoreType.DMA((NC,)),
            pltpu.SemaphoreType.DMA((NC,)),
            pltpu.SemaphoreType.DMA((NC,)),
            pltpu.SemaphoreType.DMA((NC,)),
            pltpu.SemaphoreType.DMA((NC,)),
            pltpu.SemaphoreType.DMA((NC,)),
            pltpu.SemaphoreType.DMA((NC,)),
            pltpu.SemaphoreType.DMA((NC,)),
            pltpu.SemaphoreType.DMA((2,)),
            pltpu.SemaphoreType.DMA,
            pltpu.SemaphoreType.DMA((NC,)),
        ],
        compiler_params=pltpu.CompilerParams(collective_id=0),
    )(safe_ids, mask_i32, maskf, E)
    return out


def kernel(ids, E):
    v_loc = E.shape[0]
    my_x = lax.axis_index("x")
    local = ids - my_x * v_loc
    mask = (local >= 0) & (local < v_loc)
    safe = jnp.where(mask, local, 0).astype(jnp.int32)
    mask_i32 = mask.astype(jnp.int32)
    maskf = mask.astype(jnp.float32)[:, None]
    return _fused_embed_allreduce(safe, mask_i32, maskf, E)
